# baseline (speedup 1.0000x reference)
import functools

import jax
import jax.numpy as jnp
from jax import lax
from jax.experimental import pallas as pl
from jax.experimental.pallas import tpu as pltpu
from jax.experimental.pallas import tpu_sc as plsc

N = 10000
D = 128
NC = 2
NS = 16
NW = NC * NS
CH = 128
N_PAD = 10240
TILE_ROWS = N_PAD // NS


def _sc_aggregate(h, eidx, zeros2, k, with_deg):
    mesh = plsc.VectorSubcoreMesh(core_axis_name="c", subcore_axis_name="s")
    out_type = [jax.ShapeDtypeStruct((NC, N_PAD, D), jnp.float32)]
    scratch = [
        pltpu.VMEM((2, CH), jnp.int32),
        pltpu.VMEM((2, CH), jnp.int32),
        pltpu.VMEM((2, CH), jnp.int32),
        pltpu.VMEM((2, CH), jnp.int32),
        pltpu.VMEM((CH, D), jnp.float32),
        pltpu.VMEM((CH, D), jnp.float32),
        pltpu.VMEM_SHARED((N_PAD, D), jnp.float32),
        pltpu.SemaphoreType.DMA,
        pltpu.SemaphoreType.DMA,
        pltpu.SemaphoreType.DMA,
        pltpu.SemaphoreType.DMA,
        pltpu.SemaphoreType.DMA,
        pltpu.SemaphoreType.DMA,
    ]
    if with_deg:
        out_type.append(jax.ShapeDtypeStruct((NC, N_PAD), jnp.float32))
        scratch += [
            pltpu.VMEM((CH,), jnp.float32),
            pltpu.VMEM((TILE_ROWS,), jnp.float32),
            pltpu.VMEM_SHARED((N_PAD,), jnp.float32),
        ]

    @functools.partial(pl.kernel, out_type=tuple(out_type), mesh=mesh,
                       scratch_types=scratch)
    def body(h_hbm, e_hbm, z_hbm, p_hbm, *rest):
        if with_deg:
            deg_hbm, ring0, ring1, ring2, ring3, rows0, rows1, acc_sh, \
                r0, r1, r2, r3, g0, g1, ones_v, degz_v, deg_sh = rest
        else:
            ring0, ring1, ring2, ring3, rows0, rows1, acc_sh, \
                r0, r1, r2, r3, g0, g1 = rest
        ring = (ring0, ring1, ring2, ring3)
        bufs = (rows0, rows1)
        gsem = (g0, g1)
        rsem = (r0, r1, r2, r3)
        c = lax.axis_index("c")
        s = lax.axis_index("s")
        wid = s * NC + c
        base = s * TILE_ROWS

        if with_deg:
            for i in range(CH // 16):
                ones_v[pl.ds(i * 16, 16)] = jnp.ones((16,), jnp.float32)
            for i in range(TILE_ROWS // 16):
                degz_v[pl.ds(i * 16, 16)] = jnp.zeros((16,), jnp.float32)
            pltpu.sync_copy(degz_v, deg_sh.at[pl.ds(base, TILE_ROWS)])

        pltpu.sync_copy(z_hbm.at[pl.ds(base, TILE_ROWS)],
                        acc_sh.at[pl.ds(base, TILE_ROWS)])

        plsc.subcore_barrier()

        for j in range(4):
            pltpu.async_copy(e_hbm.at[wid, j], ring[j], rsem[j])
        for b in range(2):
            pltpu.make_async_copy(e_hbm.at[wid, b], ring[b],
                                  rsem[b]).wait()
            pltpu.async_copy(h_hbm.at[ring[b].at[0]], bufs[b], gsem[b])

        def group(g, carry):
            j0 = 4 * g
            for b in range(4):
                j = j0 + b
                s2 = (b + 2) % 4
                rb = b % 2
                pltpu.make_async_copy(e_hbm.at[wid, j + 2], ring[s2],
                                      rsem[s2]).wait()
                pltpu.make_async_copy(h_hbm.at[ring[b].at[0]], bufs[rb],
                                      gsem[rb]).wait()
                pltpu.sync_copy(bufs[rb], acc_sh.at[ring[b].at[1]],
                                add=True)
                if with_deg:
                    pltpu.sync_copy(ones_v, deg_sh.at[ring[b].at[1]],
                                    add=True)
                pltpu.async_copy(h_hbm.at[ring[s2].at[0]], bufs[rb],
                                 gsem[rb])
                pltpu.async_copy(e_hbm.at[wid, j + 4], ring[b], rsem[b])
            return carry

        lax.fori_loop(0, k // 4, group, 0)

        pltpu.make_async_copy(e_hbm.at[wid, k + 2], ring[2],
                              rsem[2]).wait()
        pltpu.make_async_copy(e_hbm.at[wid, k + 3], ring[3],
                              rsem[3]).wait()
        pltpu.make_async_copy(h_hbm.at[ring[0].at[0]], rows0, gsem[0]).wait()
        pltpu.make_async_copy(h_hbm.at[ring[1].at[0]], rows1, gsem[1]).wait()

        plsc.subcore_barrier()

        pltpu.sync_copy(acc_sh.at[pl.ds(base, TILE_ROWS)],
                        p_hbm.at[c, pl.ds(base, TILE_ROWS)])
        if with_deg:
            pltpu.sync_copy(deg_sh.at[pl.ds(base, TILE_ROWS)],
                            deg_hbm.at[c, pl.ds(base, TILE_ROWS)])

    return body(h, eidx, zeros2)


def _tc_init(x, w):
    def body(x_ref, w_ref, o_ref):
        o_ref[...] = jnp.dot(x_ref[...], w_ref[...],
                             preferred_element_type=jnp.float32)

    return pl.pallas_call(
        body,
        grid=(10,),
        in_specs=[
            pl.BlockSpec((1000, D), lambda i: (i, 0)),
            pl.BlockSpec((D, D), lambda i: (0, 0)),
        ],
        out_specs=pl.BlockSpec((1000, D), lambda i: (i, 0)),
        out_shape=jax.ShapeDtypeStruct((N, D), jnp.float32),
    )(x, w)


def _tc_combine(p, deg3, h, w, b, w_out=None):
    def body(p_ref, deg_ref, h_ref, w_ref, b_ref, *rest):
        if w_out is None:
            o_ref = rest[0]
        else:
            wo_ref, o_ref = rest
        agg = p_ref[0] + p_ref[1]
        degs = deg_ref[0] + deg_ref[1]
        z = agg / (degs + 1.0) + h_ref[...]
        hn = jnp.maximum(
            jnp.dot(z, w_ref[...], preferred_element_type=jnp.float32)
            + b_ref[...], 0.0)
        if w_out is None:
            o_ref[...] = hn
        else:
            o_ref[...] = jnp.dot(hn, wo_ref[...],
                                 preferred_element_type=jnp.float32)

    in_specs = [
        pl.BlockSpec((NC, 1000, D), lambda i: (0, i, 0)),
        pl.BlockSpec((NC, 1000, 1), lambda i: (0, i, 0)),
        pl.BlockSpec((1000, D), lambda i: (i, 0)),
        pl.BlockSpec((D, D), lambda i: (0, 0)),
        pl.BlockSpec((1, D), lambda i: (0, 0)),
    ]
    args = [p, deg3, h, w, b.reshape(1, D)]
    if w_out is not None:
        in_specs.append(pl.BlockSpec((D, D), lambda i: (0, 0)))
        args.append(w_out)

    return pl.pallas_call(
        body,
        grid=(10,),
        in_specs=in_specs,
        out_specs=pl.BlockSpec((1000, D), lambda i: (i, 0)),
        out_shape=jax.ShapeDtypeStruct((N, D), jnp.float32),
    )(*args)


def kernel(x, edge_index, W_in, W_layers, b_layers, W_out):
    E = edge_index.shape[1]
    k = -(-E // (NW * CH))
    k += (-k) % 4
    e_pad = NW * k * CH

    dst = edge_index[0].astype(jnp.int32)
    src = edge_index[1].astype(jnp.int32)
    src_p = jnp.concatenate([src, jnp.zeros((e_pad - E,), jnp.int32)])
    dst_p = jnp.concatenate([dst, jnp.full((e_pad - E,), N, jnp.int32)])
    eidx = jnp.stack([src_p.reshape(NW, k, CH), dst_p.reshape(NW, k, CH)],
                     axis=2)
    pad_chunks = jnp.tile(
        jnp.stack([jnp.zeros((CH,), jnp.int32),
                   jnp.full((CH,), N, jnp.int32)])[None, None],
        (NW, 4, 1, 1))
    eidx = jnp.concatenate([eidx, pad_chunks], axis=1)
    zeros2 = jnp.zeros((N_PAD, D), jnp.float32)

    h = _tc_init(x, W_in)

    p0, deg0 = _sc_aggregate(h, eidx, zeros2, k, with_deg=True)
    deg3 = deg0[:, :, None]
    h = _tc_combine(p0, deg3, h, W_layers[0], b_layers[0])

    (p1,) = _sc_aggregate(h, eidx, zeros2, k, with_deg=False)
    out = _tc_combine(p1, deg3, h, W_layers[1], b_layers[1], w_out=W_out)
    return out


# speedup vs baseline: 1.6192x; 1.6192x over previous
import functools

import jax
import jax.numpy as jnp
from jax import lax
from jax.experimental import pallas as pl
from jax.experimental.pallas import tpu as pltpu
from jax.experimental.pallas import tpu_sc as plsc

N = 10000
D = 128
NC = 2
NS = 16
NW = NC * NS
CH = 128
N_PAD = 10240
TILE_ROWS = N_PAD // NS


def _sc_aggregate(h, eidx, zeros2, k, with_deg):
    mesh = plsc.VectorSubcoreMesh(core_axis_name="c", subcore_axis_name="s")
    Q = k // 4
    out_type = [jax.ShapeDtypeStruct((NC, N_PAD, D), jnp.float32)]
    scratch = [
        pltpu.VMEM((Q, 2, CH), jnp.int32),
        pltpu.VMEM((Q, 2, CH), jnp.int32),
        pltpu.VMEM((CH, D), jnp.float32),
        pltpu.VMEM((CH, D), jnp.float32),
        pltpu.VMEM_SHARED((N_PAD, D), jnp.float32),
        pltpu.SemaphoreType.DMA,
        pltpu.SemaphoreType.DMA,
        pltpu.SemaphoreType.DMA,
        pltpu.SemaphoreType.DMA,
    ]
    if with_deg:
        out_type.append(jax.ShapeDtypeStruct((NC, N_PAD), jnp.float32))
        scratch += [
            pltpu.VMEM((CH,), jnp.float32),
            pltpu.VMEM((TILE_ROWS,), jnp.float32),
            pltpu.VMEM_SHARED((N_PAD,), jnp.float32),
        ]

    @functools.partial(pl.kernel, out_type=tuple(out_type), mesh=mesh,
                       scratch_types=scratch)
    def body(h_hbm, e_hbm, z_hbm, p_hbm, *rest):
        if with_deg:
            deg_hbm, idx0, idx1, rows0, rows1, acc_sh, i0, i1, g0, g1, \
                ones_v, degz_v, deg_sh = rest
        else:
            idx0, idx1, rows0, rows1, acc_sh, i0, i1, g0, g1 = rest
        ibufs = (idx0, idx1)
        isem = (i0, i1)
        bufs = (rows0, rows1)
        gsem = (g0, g1)
        c = lax.axis_index("c")
        s = lax.axis_index("s")
        wid = s * NC + c
        base = s * TILE_ROWS

        if with_deg:
            for i in range(CH // 16):
                ones_v[pl.ds(i * 16, 16)] = jnp.ones((16,), jnp.float32)
            for i in range(TILE_ROWS // 16):
                degz_v[pl.ds(i * 16, 16)] = jnp.zeros((16,), jnp.float32)
            pltpu.sync_copy(degz_v, deg_sh.at[pl.ds(base, TILE_ROWS)])

        pltpu.sync_copy(z_hbm.at[pl.ds(base, TILE_ROWS)],
                        acc_sh.at[pl.ds(base, TILE_ROWS)])

        plsc.subcore_barrier()

        def chunk(ibuf, t, b):
            pltpu.make_async_copy(h_hbm.at[ibuf.at[t, 0]], bufs[b],
                                  gsem[b]).wait()
            pltpu.sync_copy(bufs[b], acc_sh.at[ibuf.at[t, 1]], add=True)
            if with_deg:
                pltpu.sync_copy(ones_v, deg_sh.at[ibuf.at[t, 1]], add=True)

        pltpu.async_copy(e_hbm.at[wid, pl.ds(0, Q)], idx0, i0)
        pltpu.make_async_copy(e_hbm.at[wid, pl.ds(0, Q)], idx0, i0).wait()
        pltpu.async_copy(h_hbm.at[idx0.at[0, 0]], rows0, g0)
        pltpu.async_copy(h_hbm.at[idx0.at[1, 0]], rows1, g1)

        for q in range(4):
            A = ibufs[q % 2]
            B = ibufs[(q + 1) % 2]
            if q < 3:
                pltpu.async_copy(e_hbm.at[wid, pl.ds((q + 1) * Q, Q)], B,
                                 isem[(q + 1) % 2])

            def pairbody(pi, carry, A=A):
                for b in range(2):
                    t = 2 * pi + b
                    chunk(A, t, b)
                    pltpu.async_copy(h_hbm.at[A.at[t + 2, 0]], bufs[b],
                                     gsem[b])
                return carry

            lax.fori_loop(0, (Q - 2) // 2, pairbody, 0)

            if q < 3:
                pltpu.make_async_copy(e_hbm.at[wid, pl.ds((q + 1) * Q, Q)],
                                      B, isem[(q + 1) % 2]).wait()
            for b in range(2):
                chunk(A, Q - 2 + b, b)
                if q < 3:
                    pltpu.async_copy(h_hbm.at[B.at[b, 0]], bufs[b], gsem[b])

        plsc.subcore_barrier()

        pltpu.sync_copy(acc_sh.at[pl.ds(base, TILE_ROWS)],
                        p_hbm.at[c, pl.ds(base, TILE_ROWS)])
        if with_deg:
            pltpu.sync_copy(deg_sh.at[pl.ds(base, TILE_ROWS)],
                            deg_hbm.at[c, pl.ds(base, TILE_ROWS)])

    return body(h, eidx, zeros2)


def _tc_init(x, w):
    def body(x_ref, w_ref, o_ref):
        o_ref[...] = jnp.dot(x_ref[...], w_ref[...],
                             preferred_element_type=jnp.float32)

    return pl.pallas_call(
        body,
        grid=(10,),
        in_specs=[
            pl.BlockSpec((1000, D), lambda i: (i, 0)),
            pl.BlockSpec((D, D), lambda i: (0, 0)),
        ],
        out_specs=pl.BlockSpec((1000, D), lambda i: (i, 0)),
        out_shape=jax.ShapeDtypeStruct((N, D), jnp.float32),
    )(x, w)


def _tc_combine(p, deg3, h, w, b, w_out=None):
    def body(p_ref, deg_ref, h_ref, w_ref, b_ref, *rest):
        if w_out is None:
            o_ref = rest[0]
        else:
            wo_ref, o_ref = rest
        agg = p_ref[0] + p_ref[1]
        degs = deg_ref[0] + deg_ref[1]
        z = agg / (degs + 1.0) + h_ref[...]
        hn = jnp.maximum(
            jnp.dot(z, w_ref[...], preferred_element_type=jnp.float32)
            + b_ref[...], 0.0)
        if w_out is None:
            o_ref[...] = hn
        else:
            o_ref[...] = jnp.dot(hn, wo_ref[...],
                                 preferred_element_type=jnp.float32)

    in_specs = [
        pl.BlockSpec((NC, 1000, D), lambda i: (0, i, 0)),
        pl.BlockSpec((NC, 1000, 1), lambda i: (0, i, 0)),
        pl.BlockSpec((1000, D), lambda i: (i, 0)),
        pl.BlockSpec((D, D), lambda i: (0, 0)),
        pl.BlockSpec((1, D), lambda i: (0, 0)),
    ]
    args = [p, deg3, h, w, b.reshape(1, D)]
    if w_out is not None:
        in_specs.append(pl.BlockSpec((D, D), lambda i: (0, 0)))
        args.append(w_out)

    return pl.pallas_call(
        body,
        grid=(10,),
        in_specs=in_specs,
        out_specs=pl.BlockSpec((1000, D), lambda i: (i, 0)),
        out_shape=jax.ShapeDtypeStruct((N, D), jnp.float32),
    )(*args)


def kernel(x, edge_index, W_in, W_layers, b_layers, W_out):
    E = edge_index.shape[1]
    k = -(-E // (NW * CH))
    k += (-k) % 8
    e_pad = NW * k * CH

    dst = edge_index[0].astype(jnp.int32)
    src = edge_index[1].astype(jnp.int32)
    src_p = jnp.concatenate([src, jnp.zeros((e_pad - E,), jnp.int32)])
    dst_p = jnp.concatenate([dst, jnp.full((e_pad - E,), N, jnp.int32)])
    eidx = jnp.stack([src_p.reshape(NW, k, CH), dst_p.reshape(NW, k, CH)],
                     axis=2)
    zeros2 = jnp.zeros((N_PAD, D), jnp.float32)

    h = _tc_init(x, W_in)

    p0, deg0 = _sc_aggregate(h, eidx, zeros2, k, with_deg=True)
    deg3 = deg0[:, :, None]
    h = _tc_combine(p0, deg3, h, W_layers[0], b_layers[0])

    (p1,) = _sc_aggregate(h, eidx, zeros2, k, with_deg=False)
    out = _tc_combine(p1, deg3, h, W_layers[1], b_layers[1], w_out=W_out)
    return out
